# baseline (speedup 1.0000x reference)
import functools

import jax
import jax.numpy as jnp
from jax import lax
from jax.experimental import pallas as pl
from jax.experimental.pallas import tpu as pltpu
from jax.experimental.pallas import tpu_sc as plsc

N = 10000
NP = 10240
E = 320000
D = 128
H = 128
C = 40
CP = 48
B = 64

NC = 2
NS = 16
NW = NC * NS
EPT = E // NW
K = 125
NCHUNK = EPT // K
IDXB = 16
NPHASE = NCHUNK // IDXB
RPT = NP // NS

R = 640
G = NP // R

_MESH = plsc.VectorSubcoreMesh(core_axis_name="c", subcore_axis_name="s",
                               num_cores=NC, num_subcores=NS)


def _sc_agg_impl(with_deg, x_hbm, src_hbm, dst_hbm, zrow_hbm, zdeg_hbm,
                 ones_hbm, agg_out, deg_out, src_v, dst_v, rows_v, ones_v,
                 agg_sh, deg_sh, gsem, ssem):
    c = lax.axis_index("c")
    s = lax.axis_index("s")
    wid = c * NS + s
    r0 = s * RPT
    pltpu.sync_copy(zrow_hbm, agg_sh.at[pl.ds(r0, RPT)])
    if with_deg:
        pltpu.sync_copy(zdeg_hbm, deg_sh.at[pl.ds(r0, RPT)])
        pltpu.sync_copy(ones_hbm, ones_v)
    plsc.subcore_barrier()

    def gather_start(j, slot):
        pltpu.async_copy(x_hbm.at[src_v.at[j]], rows_v.at[slot], gsem)

    def gather_wait(slot):
        pltpu.make_async_copy(x_hbm.at[pl.ds(0, K)], rows_v.at[slot],
                              gsem).wait()

    def scatter_start(j, slot):
        pltpu.async_copy(rows_v.at[slot], agg_sh.at[dst_v.at[j]], ssem,
                         add=True)
        if with_deg:
            pltpu.async_copy(ones_v, deg_sh.at[dst_v.at[j]], ssem, add=True)

    def scatter_wait(slot):
        pltpu.make_async_copy(rows_v.at[slot], agg_sh.at[pl.ds(0, K)],
                              ssem).wait()
        if with_deg:
            pltpu.make_async_copy(ones_v, deg_sh.at[pl.ds(0, K)],
                                  ssem).wait()

    def phase(ph, carry):
        pltpu.sync_copy(src_hbm.at[wid, pl.ds(ph * IDXB, IDXB)], src_v)
        pltpu.sync_copy(dst_hbm.at[wid, pl.ds(ph * IDXB, IDXB)], dst_v)
        gather_start(0, 0)
        gather_wait(0)
        scatter_start(0, 0)
        gather_start(1, 1)

        def step(j, c2):
            b = j % 2
            nb = 1 - b
            gather_wait(b)
            scatter_wait(nb)
            scatter_start(j, b)
            gather_start(j + 1, nb)
            return c2

        lax.fori_loop(1, IDXB - 1, step, 0)
        last = (IDXB - 1) % 2
        gather_wait(last)
        scatter_wait(1 - last)
        scatter_start(IDXB - 1, last)
        scatter_wait(last)
        return carry

    lax.fori_loop(0, NPHASE, phase, 0)

    plsc.subcore_barrier()
    pltpu.sync_copy(agg_sh.at[pl.ds(r0, RPT)], agg_out.at[c, pl.ds(r0, RPT)])
    if with_deg:
        pltpu.sync_copy(deg_sh.at[pl.ds(r0, RPT)],
                        deg_out.at[c, pl.ds(r0, RPT)])


def _sc_aggregate(x, src3, dst3, width, with_deg):
    zrow = jnp.zeros((RPT, width), jnp.float32)
    zdeg = jnp.zeros((RPT, 16), jnp.float32)
    ones = jnp.ones((K, 16), jnp.float32)
    if with_deg:
        def body(x_h, s_h, d_h, zr_h, zd_h, on_h, agg_o, deg_o,
                 src_v, dst_v, rows_v, ones_v, agg_sh, deg_sh, gsem, ssem):
            _sc_agg_impl(True, x_h, s_h, d_h, zr_h, zd_h, on_h, agg_o, deg_o,
                         src_v, dst_v, rows_v, ones_v, agg_sh, deg_sh, gsem,
                         ssem)
        out_type = [
            jax.ShapeDtypeStruct((NC, NP, width), jnp.float32),
            jax.ShapeDtypeStruct((NC, NP, 16), jnp.float32),
        ]
        shared = [
            pltpu.VMEM_SHARED((NP, width), jnp.float32),
            pltpu.VMEM_SHARED((NP, 16), jnp.float32),
        ]
    else:
        def body(x_h, s_h, d_h, zr_h, zd_h, on_h, agg_o,
                 src_v, dst_v, rows_v, ones_v, agg_sh, gsem, ssem):
            _sc_agg_impl(False, x_h, s_h, d_h, zr_h, zd_h, on_h, agg_o, None,
                         src_v, dst_v, rows_v, ones_v, agg_sh, None, gsem,
                         ssem)
        out_type = [jax.ShapeDtypeStruct((NC, NP, width), jnp.float32)]
        shared = [pltpu.VMEM_SHARED((NP, width), jnp.float32)]
    return pl.kernel(
        body,
        out_type=out_type,
        mesh=_MESH,
        compiler_params=pltpu.CompilerParams(use_tc_tiling_on_sc=False),
        scratch_types=[
            pltpu.VMEM((IDXB, K), jnp.int32),
            pltpu.VMEM((IDXB, K), jnp.int32),
            pltpu.VMEM((2, K, width), jnp.float32),
            pltpu.VMEM((K, 16), jnp.float32),
            *shared,
            pltpu.SemaphoreType.DMA,
            pltpu.SemaphoreType.DMA,
        ],
    )(x, src3, dst3, zrow, zdeg, ones)


def _tc1_body(ap, dp, xb, w1l, w1r, b1r, w2l, w2r, b2r, p_out, q_out, inv_out):
    deg = dp[0, :, 0:1] + dp[1, :, 0:1]
    inv = 1.0 / jnp.maximum(deg, 1.0)
    mean = (ap[0] + ap[1]) * inv
    h = jnp.dot(mean, w1l[...], preferred_element_type=jnp.float32)
    h = h + jnp.dot(xb[...], w1r[...], preferred_element_type=jnp.float32)
    h = jnp.maximum(h + b1r[...], 0.0)
    p_out[...] = jnp.dot(h, w2l[...], preferred_element_type=jnp.float32)
    q_out[...] = jnp.dot(h, w2r[...], preferred_element_type=jnp.float32) + b2r[...]
    inv_out[...] = jnp.broadcast_to(inv, inv_out.shape)


def _tc1(ap, dp, xp, w1lt, w1rt, b1r, w2lt, w2rt, b2r):
    row = lambda i: (i, 0)
    full = lambda i: (0, 0)
    return pl.pallas_call(
        _tc1_body,
        grid=(G,),
        in_specs=[
            pl.BlockSpec((NC, R, D), lambda i: (0, i, 0)),
            pl.BlockSpec((NC, R, 16), lambda i: (0, i, 0)),
            pl.BlockSpec((R, D), row),
            pl.BlockSpec((D, H), full),
            pl.BlockSpec((D, H), full),
            pl.BlockSpec((1, H), full),
            pl.BlockSpec((H, CP), full),
            pl.BlockSpec((H, CP), full),
            pl.BlockSpec((1, CP), full),
        ],
        out_specs=[
            pl.BlockSpec((R, CP), row),
            pl.BlockSpec((R, CP), row),
            pl.BlockSpec((R, 16), row),
        ],
        out_shape=[
            jax.ShapeDtypeStruct((NP, CP), jnp.float32),
            jax.ShapeDtypeStruct((NP, CP), jnp.float32),
            jax.ShapeDtypeStruct((NP, 16), jnp.float32),
        ],
    )(ap, dp, xp, w1lt, w1rt, b1r, w2lt, w2rt, b2r)


def _tc2_body(gp, qb, invb, b3, out_ref, acc, cnt):
    i = pl.program_id(0)

    @pl.when(i == 0)
    def _():
        acc[...] = jnp.zeros_like(acc)
        cnt[...] = jnp.zeros_like(cnt)

    h2 = (gp[0] + gp[1]) * invb[:, 0:1] + qb[...]
    brow = b3[0]
    m = (lax.broadcasted_iota(jnp.int32, (B, R), 0) == brow).astype(jnp.float32)
    acc[...] += jnp.dot(m, h2, preferred_element_type=jnp.float32)
    cnt[:, 0:1] += jnp.sum(m, axis=1, keepdims=True)

    pooled = acc[...] / jnp.maximum(cnt[:, 0:1], 1.0)
    col = lax.broadcasted_iota(jnp.int32, (B, CP), 1)
    xm = jnp.where(col < C, pooled, -jnp.inf)
    mx = jnp.max(xm, axis=1, keepdims=True)
    lse = jnp.log(jnp.sum(jnp.exp(xm - mx), axis=1, keepdims=True))
    out_ref[...] = xm - mx - lse


def _tc2(gp, q, inv, batch3):
    row = lambda i: (i, 0)
    return pl.pallas_call(
        _tc2_body,
        grid=(G,),
        in_specs=[
            pl.BlockSpec((NC, R, CP), lambda i: (0, i, 0)),
            pl.BlockSpec((R, CP), row),
            pl.BlockSpec((R, 16), row),
            pl.BlockSpec((1, 1, R), lambda i: (i, 0, 0)),
        ],
        out_specs=pl.BlockSpec((B, CP), lambda i: (0, 0)),
        out_shape=jax.ShapeDtypeStruct((B, CP), jnp.float32),
        scratch_shapes=[
            pltpu.VMEM((B, CP), jnp.float32),
            pltpu.VMEM((B, 128), jnp.float32),
        ],
    )(gp, q, inv, batch3)


def kernel(x, edge_index, batch, W1_l, W1_r, b1, W2_l, W2_r, b2):
    xp = jnp.pad(x, ((0, NP - N), (0, 0)))
    batch_pad = jnp.pad(batch, (0, NP - N), constant_values=B)
    src3 = edge_index[0].reshape(NW, NCHUNK, K)
    dst3 = edge_index[1].reshape(NW, NCHUNK, K)

    agg_part, deg_part = _sc_aggregate(xp, src3, dst3, D, True)

    w1lt = W1_l.T
    w1rt = W1_r.T
    b1r = b1.reshape(1, H)
    w2lt = jnp.pad(W2_l.T, ((0, 0), (0, CP - C)))
    w2rt = jnp.pad(W2_r.T, ((0, 0), (0, CP - C)))
    b2r = jnp.pad(b2, (0, CP - C)).reshape(1, CP)

    p, q, inv = _tc1(agg_part, deg_part, xp, w1lt, w1rt, b1r, w2lt, w2rt, b2r)

    (agg2_part,) = _sc_aggregate(p, src3, dst3, CP, False)

    batch3 = batch_pad.reshape(G, 1, R)
    out = _tc2(agg2_part, q, inv, batch3)
    return out[:, :C]


# speedup vs baseline: 1.0803x; 1.0803x over previous
import functools

import jax
import jax.numpy as jnp
from jax import lax
from jax.experimental import pallas as pl
from jax.experimental.pallas import tpu as pltpu
from jax.experimental.pallas import tpu_sc as plsc

N = 10000
NP = 10240
E = 320000
D = 128
H = 128
C = 40
CP = 48
B = 64

NC = 2
NS = 16
NW = NC * NS
EPT = E // NW
K = 125
NCHUNK = EPT // K
IDXB = 16
NPHASE = NCHUNK // IDXB
RPT = NP // NS

R = 640
G = NP // R

_MESH = plsc.VectorSubcoreMesh(core_axis_name="c", subcore_axis_name="s",
                               num_cores=NC, num_subcores=NS)


def _sc_agg_impl(with_deg, x_hbm, src_hbm, dst_hbm, zrow_hbm, zdeg_hbm,
                 ones_hbm, agg_out, deg_out, src_v, dst_v, rows_v, ones_v,
                 agg_sh, deg_sh, x_sh, gsem, ssem):
    c = lax.axis_index("c")
    s = lax.axis_index("s")
    wid = c * NS + s
    r0 = s * RPT
    pltpu.sync_copy(zrow_hbm, agg_sh.at[pl.ds(r0, RPT)])
    if with_deg:
        pltpu.sync_copy(zdeg_hbm, deg_sh.at[pl.ds(r0, RPT)])
        pltpu.sync_copy(ones_hbm, ones_v)
    if x_sh is not None:
        pltpu.sync_copy(x_hbm.at[pl.ds(r0, RPT)], x_sh.at[pl.ds(r0, RPT)])
    plsc.subcore_barrier()
    x_tab = x_hbm if x_sh is None else x_sh

    def gather_start(j, slot):
        pltpu.async_copy(x_tab.at[src_v.at[j]], rows_v.at[slot], gsem)

    def gather_wait(slot):
        pltpu.make_async_copy(x_hbm.at[pl.ds(0, K)], rows_v.at[slot],
                              gsem).wait()

    def scatter_start(j, slot):
        pltpu.async_copy(rows_v.at[slot], agg_sh.at[dst_v.at[j]], ssem,
                         add=True)
        if with_deg:
            pltpu.async_copy(ones_v, deg_sh.at[dst_v.at[j]], ssem, add=True)

    def scatter_wait(slot):
        pltpu.make_async_copy(rows_v.at[slot], agg_sh.at[pl.ds(0, K)],
                              ssem).wait()
        if with_deg:
            pltpu.make_async_copy(ones_v, deg_sh.at[pl.ds(0, K)],
                                  ssem).wait()

    def phase(ph, carry):
        pltpu.sync_copy(src_hbm.at[wid, pl.ds(ph * IDXB, IDXB)], src_v)
        pltpu.sync_copy(dst_hbm.at[wid, pl.ds(ph * IDXB, IDXB)], dst_v)
        gather_start(0, 0)
        gather_wait(0)
        scatter_start(0, 0)
        gather_start(1, 1)

        def step(j, c2):
            b = j % 2
            nb = 1 - b
            gather_wait(b)
            scatter_wait(nb)
            scatter_start(j, b)
            gather_start(j + 1, nb)
            return c2

        lax.fori_loop(1, IDXB - 1, step, 0)
        last = (IDXB - 1) % 2
        gather_wait(last)
        scatter_wait(1 - last)
        scatter_start(IDXB - 1, last)
        scatter_wait(last)
        return carry

    lax.fori_loop(0, NPHASE, phase, 0)

    plsc.subcore_barrier()
    pltpu.sync_copy(agg_sh.at[pl.ds(r0, RPT)], agg_out.at[c, pl.ds(r0, RPT)])
    if with_deg:
        pltpu.sync_copy(deg_sh.at[pl.ds(r0, RPT)],
                        deg_out.at[c, pl.ds(r0, RPT)])


def _sc_aggregate(x, src3, dst3, width, with_deg):
    zrow = jnp.zeros((RPT, width), jnp.float32)
    zdeg = jnp.zeros((RPT, 16), jnp.float32)
    ones = jnp.ones((K, 16), jnp.float32)
    if with_deg:
        def body(x_h, s_h, d_h, zr_h, zd_h, on_h, agg_o, deg_o,
                 src_v, dst_v, rows_v, ones_v, agg_sh, deg_sh, gsem, ssem):
            _sc_agg_impl(True, x_h, s_h, d_h, zr_h, zd_h, on_h, agg_o, deg_o,
                         src_v, dst_v, rows_v, ones_v, agg_sh, deg_sh, None,
                         gsem, ssem)
        out_type = [
            jax.ShapeDtypeStruct((NC, NP, width), jnp.float32),
            jax.ShapeDtypeStruct((NC, NP, 16), jnp.float32),
        ]
        shared = [
            pltpu.VMEM_SHARED((NP, width), jnp.float32),
            pltpu.VMEM_SHARED((NP, 16), jnp.float32),
        ]
    else:
        def body(x_h, s_h, d_h, zr_h, zd_h, on_h, agg_o,
                 src_v, dst_v, rows_v, ones_v, agg_sh, x_sh, gsem, ssem):
            _sc_agg_impl(False, x_h, s_h, d_h, zr_h, zd_h, on_h, agg_o, None,
                         src_v, dst_v, rows_v, ones_v, agg_sh, None, x_sh,
                         gsem, ssem)
        out_type = [jax.ShapeDtypeStruct((NC, NP, width), jnp.float32)]
        shared = [
            pltpu.VMEM_SHARED((NP, width), jnp.float32),
            pltpu.VMEM_SHARED((NP, width), jnp.float32),
        ]
    return pl.kernel(
        body,
        out_type=out_type,
        mesh=_MESH,
        compiler_params=pltpu.CompilerParams(use_tc_tiling_on_sc=False),
        scratch_types=[
            pltpu.VMEM((IDXB, K), jnp.int32),
            pltpu.VMEM((IDXB, K), jnp.int32),
            pltpu.VMEM((2, K, width), jnp.float32),
            pltpu.VMEM((K, 16), jnp.float32),
            *shared,
            pltpu.SemaphoreType.DMA,
            pltpu.SemaphoreType.DMA,
        ],
    )(x, src3, dst3, zrow, zdeg, ones)


def _tc1_body(ap, dp, xb, w1l, w1r, b1r, w2l, w2r, b2r, p_out, q_out, inv_out):
    deg = dp[0, :, 0:1] + dp[1, :, 0:1]
    inv = 1.0 / jnp.maximum(deg, 1.0)
    mean = (ap[0] + ap[1]) * inv
    h = jnp.dot(mean, w1l[...], preferred_element_type=jnp.float32)
    h = h + jnp.dot(xb[...], w1r[...], preferred_element_type=jnp.float32)
    h = jnp.maximum(h + b1r[...], 0.0)
    p_out[...] = jnp.dot(h, w2l[...], preferred_element_type=jnp.float32)
    q_out[...] = jnp.dot(h, w2r[...], preferred_element_type=jnp.float32) + b2r[...]
    inv_out[...] = jnp.broadcast_to(inv, inv_out.shape)


def _tc1(ap, dp, xp, w1lt, w1rt, b1r, w2lt, w2rt, b2r):
    row = lambda i: (i, 0)
    full = lambda i: (0, 0)
    return pl.pallas_call(
        _tc1_body,
        grid=(G,),
        in_specs=[
            pl.BlockSpec((NC, R, D), lambda i: (0, i, 0)),
            pl.BlockSpec((NC, R, 16), lambda i: (0, i, 0)),
            pl.BlockSpec((R, D), row),
            pl.BlockSpec((D, H), full),
            pl.BlockSpec((D, H), full),
            pl.BlockSpec((1, H), full),
            pl.BlockSpec((H, CP), full),
            pl.BlockSpec((H, CP), full),
            pl.BlockSpec((1, CP), full),
        ],
        out_specs=[
            pl.BlockSpec((R, CP), row),
            pl.BlockSpec((R, CP), row),
            pl.BlockSpec((R, 16), row),
        ],
        out_shape=[
            jax.ShapeDtypeStruct((NP, CP), jnp.float32),
            jax.ShapeDtypeStruct((NP, CP), jnp.float32),
            jax.ShapeDtypeStruct((NP, 16), jnp.float32),
        ],
    )(ap, dp, xp, w1lt, w1rt, b1r, w2lt, w2rt, b2r)


def _tc2_body(gp, qb, invb, b3, out_ref, acc, cnt):
    i = pl.program_id(0)

    @pl.when(i == 0)
    def _():
        acc[...] = jnp.zeros_like(acc)
        cnt[...] = jnp.zeros_like(cnt)

    h2 = (gp[0] + gp[1]) * invb[:, 0:1] + qb[...]
    brow = b3[0]
    m = (lax.broadcasted_iota(jnp.int32, (B, R), 0) == brow).astype(jnp.float32)
    acc[...] += jnp.dot(m, h2, preferred_element_type=jnp.float32)
    cnt[:, 0:1] += jnp.sum(m, axis=1, keepdims=True)

    pooled = acc[...] / jnp.maximum(cnt[:, 0:1], 1.0)
    col = lax.broadcasted_iota(jnp.int32, (B, CP), 1)
    xm = jnp.where(col < C, pooled, -jnp.inf)
    mx = jnp.max(xm, axis=1, keepdims=True)
    lse = jnp.log(jnp.sum(jnp.exp(xm - mx), axis=1, keepdims=True))
    out_ref[...] = xm - mx - lse


def _tc2(gp, q, inv, batch3):
    row = lambda i: (i, 0)
    return pl.pallas_call(
        _tc2_body,
        grid=(G,),
        in_specs=[
            pl.BlockSpec((NC, R, CP), lambda i: (0, i, 0)),
            pl.BlockSpec((R, CP), row),
            pl.BlockSpec((R, 16), row),
            pl.BlockSpec((1, 1, R), lambda i: (i, 0, 0)),
        ],
        out_specs=pl.BlockSpec((B, CP), lambda i: (0, 0)),
        out_shape=jax.ShapeDtypeStruct((B, CP), jnp.float32),
        scratch_shapes=[
            pltpu.VMEM((B, CP), jnp.float32),
            pltpu.VMEM((B, 128), jnp.float32),
        ],
    )(gp, q, inv, batch3)


def kernel(x, edge_index, batch, W1_l, W1_r, b1, W2_l, W2_r, b2):
    xp = jnp.pad(x, ((0, NP - N), (0, 0)))
    batch_pad = jnp.pad(batch, (0, NP - N), constant_values=B)
    src3 = edge_index[0].reshape(NW, NCHUNK, K)
    dst3 = edge_index[1].reshape(NW, NCHUNK, K)

    agg_part, deg_part = _sc_aggregate(xp, src3, dst3, D, True)

    w1lt = W1_l.T
    w1rt = W1_r.T
    b1r = b1.reshape(1, H)
    w2lt = jnp.pad(W2_l.T, ((0, 0), (0, CP - C)))
    w2rt = jnp.pad(W2_r.T, ((0, 0), (0, CP - C)))
    b2r = jnp.pad(b2, (0, CP - C)).reshape(1, CP)

    p, q, inv = _tc1(agg_part, deg_part, xp, w1lt, w1rt, b1r, w2lt, w2rt, b2r)

    (agg2_part,) = _sc_aggregate(p, src3, dst3, CP, False)

    batch3 = batch_pad.reshape(G, 1, R)
    out = _tc2(agg2_part, q, inv, batch3)
    return out[:, :C]
